# baseline (speedup 1.0000x reference)
import functools

import jax
import jax.numpy as jnp
from jax import lax
from jax.experimental import pallas as pl
from jax.experimental.pallas import tpu as pltpu
from jax.experimental.pallas import tpu_sc as plsc

BATCH = 16384
EMBED_DIM = 64
DENSE_IN = 384
OUT_DIM = 4 * EMBED_DIM

NC = 2
NS = 16
NW = NC * NS
ROWS_W = BATCH // NW
CHUNK = 64
N_CHUNKS = ROWS_W // CHUNK


def _matmul_body(x_ref, w_ref, b_ref, o_ref):
    o_ref[...] = (
        jnp.dot(x_ref[...], w_ref[...], preferred_element_type=jnp.float32)
        + b_ref[...]
    )


def _dense_matmul(x, W, b):
    block_rows = 1024
    return pl.pallas_call(
        _matmul_body,
        grid=(BATCH // block_rows,),
        in_specs=[
            pl.BlockSpec((block_rows, DENSE_IN), lambda i: (i, 0)),
            pl.BlockSpec((DENSE_IN, EMBED_DIM), lambda i: (0, 0)),
            pl.BlockSpec((1, EMBED_DIM), lambda i: (0, 0)),
        ],
        out_specs=pl.BlockSpec((block_rows, EMBED_DIM), lambda i: (i, 0)),
        out_shape=jax.ShapeDtypeStruct((BATCH, EMBED_DIM), jnp.float32),
    )(x, W, b.reshape(1, EMBED_DIM))


def _sc_body(ablk, aoff, dblk, doff, pblk, poff, atab, dtab, ptab, dense,
             out, blks_v, offs_v, abuf, dbuf, pbuf, xbuf, asm, gsem, xsem,
             wsem):
    wid = lax.axis_index("s") * NC + lax.axis_index("c")
    base = wid * ROWS_W
    r0 = wid * N_CHUNKS

    pltpu.sync_copy(ablk.at[pl.ds(r0, N_CHUNKS)], blks_v.at[0])
    pltpu.sync_copy(dblk.at[pl.ds(r0, N_CHUNKS)], blks_v.at[1])
    pltpu.sync_copy(pblk.at[pl.ds(r0, N_CHUNKS)], blks_v.at[2])
    pltpu.sync_copy(aoff.at[pl.ds(r0, N_CHUNKS)], offs_v.at[0])
    pltpu.sync_copy(doff.at[pl.ds(r0, N_CHUNKS)], offs_v.at[1])
    pltpu.sync_copy(poff.at[pl.ds(r0, N_CHUNKS)], offs_v.at[2])

    lane = lax.iota(jnp.int32, 16)
    tabs = ((atab, abuf, 0), (dtab, dbuf, 1), (ptab, pbuf, 2))

    def fire_blocks(j):
        def group(g, carry):
            for tab, buf, t in tabs:
                vec = blks_v.at[t].at[j][pl.ds(g * 16, 16)]
                for l in range(16):
                    s = pl.multiple_of(vec[l], 8)
                    pltpu.async_copy(
                        tab.at[pl.ds(s, 8)],
                        buf.at[pl.ds((g * 16 + l) * 8, 8)],
                        gsem,
                    )
            return carry

        lax.fori_loop(0, CHUNK // 16, group, 0)

    def drain_blocks():
        for tab, buf, _ in tabs:
            pltpu.make_async_copy(
                tab.at[pl.ds(0, 8 * CHUNK)], buf, gsem
            ).wait()

    def assemble(j):
        def row_body(i, carry):
            i_vec = jnp.zeros((16,), jnp.int32) + i
            for t, (_, buf, tt) in zip((0, 1, 2), tabs):
                off_vec = plsc.load_gather(offs_v.at[tt].at[j], [i_vec])
                pos_vec = i_vec * 8 + off_vec
                for c in range(EMBED_DIM // 16):
                    col = lane + c * 16
                    asm[i, pl.ds(t * EMBED_DIM + c * 16, 16)] = (
                        plsc.load_gather(buf, [pos_vec, col])
                    )
            for c in range(EMBED_DIM // 16):
                asm[i, pl.ds(3 * EMBED_DIM + c * 16, 16)] = (
                    xbuf[i, pl.ds(c * 16, 16)]
                )
            return carry

        lax.fori_loop(0, CHUNK, row_body, 0)

    for j in range(N_CHUNKS):
        fire_blocks(j)
        hx = pltpu.async_copy(
            dense.at[pl.ds(base + j * CHUNK, CHUNK)], xbuf, xsem
        )
        drain_blocks()
        hx.wait()
        assemble(j)
        pltpu.sync_copy(asm, out.at[pl.ds(base + j * CHUNK, CHUNK)])


_sc_assemble = functools.partial(
    pl.kernel,
    out_type=jax.ShapeDtypeStruct((BATCH, OUT_DIM), jnp.float32),
    mesh=plsc.VectorSubcoreMesh(
        core_axis_name="c", subcore_axis_name="s", num_cores=NC, num_subcores=NS
    ),
    scratch_types=[
        pltpu.VMEM((3, N_CHUNKS, CHUNK), jnp.int32),
        pltpu.VMEM((3, N_CHUNKS, CHUNK), jnp.int32),
        pltpu.VMEM((8 * CHUNK, EMBED_DIM), jnp.float32),
        pltpu.VMEM((8 * CHUNK, EMBED_DIM), jnp.float32),
        pltpu.VMEM((8 * CHUNK, EMBED_DIM), jnp.float32),
        pltpu.VMEM((CHUNK, EMBED_DIM), jnp.float32),
        pltpu.VMEM((CHUNK, OUT_DIM), jnp.float32),
        pltpu.SemaphoreType.DMA,
        pltpu.SemaphoreType.DMA,
        pltpu.SemaphoreType.DMA,
    ],
    compiler_params=pltpu.CompilerParams(
        needs_layout_passes=False, use_tc_tiling_on_sc=False
    ),
)(_sc_body)


def _split_idx(i):
    i = i.astype(jnp.int32)
    blk = (i & ~7).reshape(BATCH // CHUNK, CHUNK)
    off = (i & 7).reshape(BATCH // CHUNK, CHUNK)
    return blk, off


def kernel(book_ids, book_authors, book_dates, book_publishers,
           book_title_embeddings, id_table, author_table, date_table,
           publisher_table, W, b):
    dense = _dense_matmul(book_title_embeddings, W, b)
    ablk, aoff = _split_idx(book_authors)
    dblk, doff = _split_idx(book_dates)
    pblk, poff = _split_idx(book_publishers)
    return _sc_assemble(
        ablk, aoff, dblk, doff, pblk, poff,
        author_table, date_table, publisher_table, dense,
    )


# speedup vs baseline: 1.0695x; 1.0695x over previous
import functools

import jax
import jax.numpy as jnp
from jax import lax
from jax.experimental import pallas as pl
from jax.experimental.pallas import tpu as pltpu
from jax.experimental.pallas import tpu_sc as plsc

BATCH = 16384
EMBED_DIM = 64
DENSE_IN = 384
OUT_DIM = 4 * EMBED_DIM

NC = 2
NS = 16
NW = NC * NS
ROWS_W = BATCH // NW
CHUNK = 32
N_CHUNKS = ROWS_W // CHUNK


def _matmul_body(x_ref, w_ref, b_ref, o_ref):
    o_ref[...] = (
        jnp.dot(x_ref[...], w_ref[...], preferred_element_type=jnp.float32)
        + b_ref[...]
    )


def _dense_matmul(x, W, b):
    block_rows = 1024
    return pl.pallas_call(
        _matmul_body,
        grid=(BATCH // block_rows,),
        in_specs=[
            pl.BlockSpec((block_rows, DENSE_IN), lambda i: (i, 0)),
            pl.BlockSpec((DENSE_IN, EMBED_DIM), lambda i: (0, 0)),
            pl.BlockSpec((1, EMBED_DIM), lambda i: (0, 0)),
        ],
        out_specs=pl.BlockSpec((block_rows, EMBED_DIM), lambda i: (i, 0)),
        out_shape=jax.ShapeDtypeStruct((BATCH, EMBED_DIM), jnp.float32),
    )(x, W, b.reshape(1, EMBED_DIM))


def _sc_body(ablk, aoff, dblk, doff, pblk, poff, atab, dtab, ptab, dense,
             out, blks_v, offs_v, abuf, dbuf, pbuf, xbuf, asm, gsem, xsem):
    wid = lax.axis_index("s") * NC + lax.axis_index("c")
    base = wid * ROWS_W
    r0 = wid * N_CHUNKS

    pltpu.sync_copy(ablk.at[pl.ds(r0, N_CHUNKS)], blks_v.at[0])
    pltpu.sync_copy(dblk.at[pl.ds(r0, N_CHUNKS)], blks_v.at[1])
    pltpu.sync_copy(pblk.at[pl.ds(r0, N_CHUNKS)], blks_v.at[2])
    pltpu.sync_copy(aoff.at[pl.ds(r0, N_CHUNKS)], offs_v.at[0])
    pltpu.sync_copy(doff.at[pl.ds(r0, N_CHUNKS)], offs_v.at[1])
    pltpu.sync_copy(poff.at[pl.ds(r0, N_CHUNKS)], offs_v.at[2])

    lane = lax.iota(jnp.int32, 16)
    tabs = ((atab, abuf, 0), (dtab, dbuf, 1), (ptab, pbuf, 2))

    def chunk_body(j, carry):
        def group(g, c2):
            for tab, buf, t in tabs:
                vec = blks_v.at[t].at[j][pl.ds(g * 16, 16)]
                for l in range(16):
                    s = pl.multiple_of(vec[l], 8)
                    pltpu.async_copy(
                        tab.at[pl.ds(s, 8)],
                        buf.at[pl.ds((g * 16 + l) * 8, 8)],
                        gsem,
                    )
            return c2

        lax.fori_loop(0, CHUNK // 16, group, 0)
        hx = pltpu.async_copy(
            dense.at[pl.ds(
                pl.multiple_of(base + j * CHUNK, 8), CHUNK
            )],
            xbuf,
            xsem,
        )
        for tab, buf, _ in tabs:
            pltpu.make_async_copy(
                tab.at[pl.ds(0, 8 * CHUNK)], buf, gsem
            ).wait()
        hx.wait()

        def row_body(i, c2):
            i_vec = jnp.zeros((16,), jnp.int32) + i
            for t, (_, buf, tt) in zip((0, 1, 2), tabs):
                off_vec = plsc.load_gather(offs_v.at[tt].at[j], [i_vec])
                pos_vec = i_vec * 8 + off_vec
                for c in range(EMBED_DIM // 16):
                    col = lane + c * 16
                    asm[i, pl.ds(t * EMBED_DIM + c * 16, 16)] = (
                        plsc.load_gather(buf, [pos_vec, col])
                    )
            for c in range(EMBED_DIM // 16):
                asm[i, pl.ds(3 * EMBED_DIM + c * 16, 16)] = (
                    xbuf[i, pl.ds(c * 16, 16)]
                )
            return c2

        lax.fori_loop(0, CHUNK, row_body, 0)
        pltpu.sync_copy(
            asm,
            out.at[pl.ds(pl.multiple_of(base + j * CHUNK, 8), CHUNK)],
        )
        return carry

    lax.fori_loop(0, N_CHUNKS, chunk_body, 0)


_sc_assemble = functools.partial(
    pl.kernel,
    out_type=jax.ShapeDtypeStruct((BATCH, OUT_DIM), jnp.float32),
    mesh=plsc.VectorSubcoreMesh(
        core_axis_name="c", subcore_axis_name="s", num_cores=NC, num_subcores=NS
    ),
    scratch_types=[
        pltpu.VMEM((3, N_CHUNKS, CHUNK), jnp.int32),
        pltpu.VMEM((3, N_CHUNKS, CHUNK), jnp.int32),
        pltpu.VMEM((8 * CHUNK, EMBED_DIM), jnp.float32),
        pltpu.VMEM((8 * CHUNK, EMBED_DIM), jnp.float32),
        pltpu.VMEM((8 * CHUNK, EMBED_DIM), jnp.float32),
        pltpu.VMEM((CHUNK, EMBED_DIM), jnp.float32),
        pltpu.VMEM((CHUNK, OUT_DIM), jnp.float32),
        pltpu.SemaphoreType.DMA,
        pltpu.SemaphoreType.DMA,
    ],
    compiler_params=pltpu.CompilerParams(needs_layout_passes=False),
)(_sc_body)


def _split_idx(i):
    i = i.astype(jnp.int32)
    blk = (i & ~7).reshape(BATCH // CHUNK, CHUNK)
    off = (i & 7).reshape(BATCH // CHUNK, CHUNK)
    return blk, off


def kernel(book_ids, book_authors, book_dates, book_publishers,
           book_title_embeddings, id_table, author_table, date_table,
           publisher_table, W, b):
    dense = _dense_matmul(book_title_embeddings, W, b)
    ablk, aoff = _split_idx(book_authors)
    dblk, doff = _split_idx(book_dates)
    pblk, poff = _split_idx(book_publishers)
    return _sc_assemble(
        ablk, aoff, dblk, doff, pblk, poff,
        author_table, date_table, publisher_table, dense,
    )
